# baseline (speedup 1.0000x reference)
import functools

import jax
import jax.numpy as jnp
from jax import lax
from jax.experimental import pallas as pl
from jax.experimental.pallas import tpu as pltpu, tpu_sc as plsc

N = 10000
D = 128
E = 320000
EL = E + N
NP = 10112
AW = 144
K = 32
NC = 2
NS = 16
NW = NC * NS
NBLK = 324
PER_TILE = NBLK * K
EP = PER_TILE * NW
EPAD = EP + 3 * K
ROWS_PER_TILE = NP // NS


def _sc_conv_body(xl_hbm, xr_hbm, src_hbm, dst_hbm, att_hbm, out_hbm,
                  src_i, dst_i, xl_rows, xr_rows, stage, att_v,
                  acc_sh, isem, glsem, grsem, ssem):
    c = lax.axis_index("c")
    s = lax.axis_index("s")
    wid = s * NC + c
    base0 = wid * PER_TILE

    pltpu.sync_copy(att_hbm, att_v)
    iota = lax.iota(jnp.int32, 16)
    lane0 = iota == 0
    perm = [jnp.bitwise_xor(iota, 1 << p) for p in range(4)]
    attc = [att_v[pl.ds(16 * g, 16)] for g in range(D // 16)]
    zero16 = jnp.zeros((16,), jnp.float32)

    for sp in range(2):
        @plsc.parallel_loop(0, K, unroll=4)
        def zrow(j, sp=sp):
            for g in range(AW // 16):
                stage[sp][j, pl.ds(16 * g, 16)] = zero16
    row0 = s * ROWS_PER_TILE
    nfull = ROWS_PER_TILE // K
    for k in range(nfull):
        pltpu.sync_copy(stage[0], acc_sh.at[pl.ds(row0 + k * K, K)])
    rem = ROWS_PER_TILE - nfull * K
    if rem:
        pltpu.sync_copy(stage[0].at[pl.ds(0, rem)],
                        acc_sh.at[pl.ds(row0 + nfull * K, rem)])
    plsc.subcore_barrier()

    def idx_issue(blk, q):
        base = base0 + blk * K
        pltpu.async_copy(src_hbm.at[pl.ds(base, K)], src_i[q], isem[q])
        pltpu.async_copy(dst_hbm.at[pl.ds(base, K)], dst_i[q], isem[q])

    def idx_wait(q):
        pltpu.make_async_copy(src_hbm.at[pl.ds(0, K)], src_i[q], isem[q]).wait()
        pltpu.make_async_copy(dst_hbm.at[pl.ds(0, K)], dst_i[q], isem[q]).wait()

    def gather_issue(q, p):
        pltpu.async_copy(xl_hbm.at[src_i[q]], xl_rows[p], glsem[p])
        pltpu.async_copy(xr_hbm.at[dst_i[q]], xr_rows[p], grsem[p])

    def gather_wait(q, p):
        pltpu.make_async_copy(xl_hbm.at[src_i[q]], xl_rows[p], glsem[p]).wait()
        pltpu.make_async_copy(xr_hbm.at[dst_i[q]], xr_rows[p], grsem[p]).wait()

    def scatter_issue(q, sp):
        pltpu.async_copy(stage[sp], acc_sh.at[dst_i[q]], ssem[sp], add=True)

    def scatter_wait(q, sp):
        pltpu.make_async_copy(stage[sp], acc_sh.at[dst_i[q]], ssem[sp]).wait()

    def compute(p):
        @plsc.parallel_loop(0, K, unroll=4)
        def edge_body(j):
            acc = zero16
            xs = []
            for g in range(D // 16):
                a = xl_rows[p][j, pl.ds(16 * g, 16)]
                b = xr_rows[p][j, pl.ds(16 * g, 16)]
                xs.append(a)
                m = a + b
                lr = jnp.maximum(m, 0.2 * m)
                acc = acc + attc[g] * lr
            for pp in range(4):
                acc = acc + acc.at[perm[pp]].get(mode="promise_in_bounds")
            exv = jnp.exp(acc)
            for g in range(D // 16):
                stage[p][j, pl.ds(16 * g, 16)] = xs[g] * exv
            stage[p][j, pl.ds(D, 16)] = jnp.where(lane0, exv, 0.0)

    idx_issue(0, 0)
    idx_issue(1, 1)
    idx_wait(0)
    gather_issue(0, 0)

    def group_body(t, _):
        for phi in range(4):
            blk = t * 4 + phi
            p = phi % 2
            q = phi % 4
            if phi < 2:
                @pl.when(t > 0)
                def _():
                    scatter_wait((phi - 2) % 4, p)
            else:
                scatter_wait(phi - 2, p)
            idx_issue(blk + 2, (phi + 2) % 4)
            gather_wait(q, p)
            compute(p)
            scatter_issue(q, p)
            idx_wait((phi + 1) % 4)
            gather_issue((phi + 1) % 4, (phi + 1) % 2)
        return 0
    lax.fori_loop(0, NBLK // 4, group_body, 0)

    scatter_wait((NBLK - 2) % 4, (NBLK - 2) % 2)
    scatter_wait((NBLK - 1) % 4, (NBLK - 1) % 2)
    gather_wait(NBLK % 4, NBLK % 2)
    idx_wait((NBLK + 1) % 4)

    plsc.subcore_barrier()
    for k in range(nfull):
        r = row0 + k * K
        pltpu.sync_copy(acc_sh.at[pl.ds(r, K)], out_hbm.at[c, pl.ds(r, K)])
    if rem:
        r = row0 + nfull * K
        pltpu.sync_copy(acc_sh.at[pl.ds(r, rem)], out_hbm.at[c, pl.ds(r, rem)])


_sc_conv = pl.kernel(
    _sc_conv_body,
    out_type=jax.ShapeDtypeStruct((NC, NP, AW), jnp.float32),
    mesh=plsc.VectorSubcoreMesh(core_axis_name="c", subcore_axis_name="s"),
    compiler_params=pltpu.CompilerParams(use_tc_tiling_on_sc=False),
    scratch_types=[
        [pltpu.VMEM((K,), jnp.int32)] * 4,
        [pltpu.VMEM((K,), jnp.int32)] * 4,
        [pltpu.VMEM((K, D), jnp.float32)] * 2,
        [pltpu.VMEM((K, D), jnp.float32)] * 2,
        [pltpu.VMEM((K, AW), jnp.float32)] * 2,
        pltpu.VMEM((D,), jnp.float32),
        pltpu.VMEM_SHARED((NP, AW), jnp.float32),
        [pltpu.SemaphoreType.DMA] * 4,
        [pltpu.SemaphoreType.DMA] * 2,
        [pltpu.SemaphoreType.DMA] * 2,
        [pltpu.SemaphoreType.DMA] * 2,
    ],
)


_MM_ROWS = 1264


def _mm_block_body(x_ref, w_ref, *o_refs):
    r = jnp.dot(x_ref[...], w_ref[...], preferred_element_type=jnp.float32)
    for i, o in enumerate(o_refs):
        o[...] = r[:, i * D:(i + 1) * D]


def _mm(xp, w, nout):
    grid = NP // _MM_ROWS
    return pl.pallas_call(
        _mm_block_body,
        grid=(grid,),
        in_specs=[
            pl.BlockSpec((_MM_ROWS, D), lambda i: (i, 0)),
            pl.BlockSpec((D, nout * D), lambda i: (0, 0)),
        ],
        out_specs=[pl.BlockSpec((_MM_ROWS, D), lambda i: (i, 0))] * nout,
        out_shape=[jax.ShapeDtypeStruct((NP, D), jnp.float32)] * nout,
    )(xp, w)


def _bn_from_acc(p, b, g, be, mask):
    m = p[0] + p[1]
    o = m[:, :D] / (m[:, D:D + 1] + 1e-16) + b
    o = jnp.where(mask, o, 0.0)
    mean = jnp.sum(o, axis=0, keepdims=True) * (1.0 / N)
    d = jnp.where(mask, o - mean, 0.0)
    var = jnp.sum(d * d, axis=0, keepdims=True) * (1.0 / N)
    return (o - mean) * lax.rsqrt(var + 1e-5) * g + be


def _epi1_body(p_ref, b_ref, g_ref, be_ref, o_ref):
    mask = lax.broadcasted_iota(jnp.int32, (NP, 1), 0) < N
    y = _bn_from_acc(p_ref[...], b_ref[...], g_ref[...], be_ref[...], mask)
    o_ref[...] = jnp.where(mask, jnp.maximum(y, 0.0), 0.0)


def _epif_body(p2_ref, b2_ref, g2_ref, be2_ref,
               p3_ref, b3_ref, g3_ref, be3_ref, o_ref):
    mask = lax.broadcasted_iota(jnp.int32, (NP, 1), 0) < N
    y2 = _bn_from_acc(p2_ref[...], b2_ref[...], g2_ref[...], be2_ref[...], mask)
    y3 = _bn_from_acc(p3_ref[...], b3_ref[...], g3_ref[...], be3_ref[...], mask)
    o_ref[...] = jnp.where(mask, jnp.maximum(y2 + y3, 0.0), 0.0)


def _epi1(acc, b, g, be):
    return pl.pallas_call(
        _epi1_body,
        out_shape=jax.ShapeDtypeStruct((NP, D), jnp.float32),
    )(acc, b.reshape(1, D), g.reshape(1, D), be.reshape(1, D))


def _epif(acc2, b2, g2, be2, acc3, b3, g3, be3):
    return pl.pallas_call(
        _epif_body,
        out_shape=jax.ShapeDtypeStruct((NP, D), jnp.float32),
    )(acc2, b2.reshape(1, D), g2.reshape(1, D), be2.reshape(1, D),
      acc3, b3.reshape(1, D), g3.reshape(1, D), be3.reshape(1, D))


def kernel(x, edge_index, Wl1, Wr1, att1, b1, g1, be1,
           Wl2, Wr2, att2, b2, g2, be2,
           Wl3, Wr3, att3, b3, g3, be3):
    f32 = jnp.float32
    xp = jnp.zeros((NP, D), f32).at[:N].set(x.astype(f32))

    src = edge_index[0].astype(jnp.int32)
    dst = edge_index[1].astype(jnp.int32)
    loop = jnp.arange(N, dtype=jnp.int32)
    padi = jnp.full((EPAD - EL,), N, jnp.int32)
    srcp = jnp.concatenate([src, loop, padi])
    dstp = jnp.concatenate([dst, loop, padi])

    w1 = jnp.concatenate([Wl1.T, Wr1.T, Wl3.T, Wr3.T], axis=1)
    xl1, xr1, xl3, xr3 = _mm(xp, w1, 4)
    acc1 = _sc_conv(xl1, xr1, srcp, dstp, att1)
    h1 = _epi1(acc1, b1, g1, be1)

    w2 = jnp.concatenate([Wl2.T, Wr2.T], axis=1)
    xl2, xr2 = _mm(h1, w2, 2)
    acc2 = _sc_conv(xl2, xr2, srcp, dstp, att2)
    acc3 = _sc_conv(xl3, xr3, srcp, dstp, att3)

    y = _epif(acc2, b2, g2, be2, acc3, b3, g3, be3)
    return y[:N]


# speedup vs baseline: 1.1448x; 1.1448x over previous
import functools

import jax
import jax.numpy as jnp
from jax import lax
from jax.experimental import pallas as pl
from jax.experimental.pallas import tpu as pltpu, tpu_sc as plsc

N = 10000
D = 128
E = 320000
EL = E + N
NP = 10112
AW = 144
K = 48
NC = 2
NS = 16
NW = NC * NS
NBLK = 216
PER_TILE = NBLK * K
EP = PER_TILE * NW
EPAD = EP + 3 * K
ROWS_PER_TILE = NP // NS


def _sc_conv_body(xl_hbm, xr_hbm, src_hbm, dst_hbm, att_hbm, out_hbm,
                  src_i, dst_i, xl_rows, xr_rows, stage, att_v,
                  acc_sh, isem, glsem, grsem, ssem):
    c = lax.axis_index("c")
    s = lax.axis_index("s")
    wid = s * NC + c
    base0 = wid * PER_TILE

    pltpu.sync_copy(att_hbm, att_v)
    iota = lax.iota(jnp.int32, 16)
    lane0 = iota == 0
    perm = [jnp.bitwise_xor(iota, 1 << p) for p in range(4)]
    attc = [att_v[pl.ds(16 * g, 16)] for g in range(D // 16)]
    zero16 = jnp.zeros((16,), jnp.float32)

    for sp in range(2):
        @plsc.parallel_loop(0, K, unroll=4)
        def zrow(j, sp=sp):
            for g in range(AW // 16):
                stage[sp][j, pl.ds(16 * g, 16)] = zero16
    row0 = s * ROWS_PER_TILE
    nfull = ROWS_PER_TILE // K
    for k in range(nfull):
        pltpu.sync_copy(stage[0], acc_sh.at[pl.ds(row0 + k * K, K)])
    rem = ROWS_PER_TILE - nfull * K
    if rem:
        pltpu.sync_copy(stage[0].at[pl.ds(0, rem)],
                        acc_sh.at[pl.ds(row0 + nfull * K, rem)])
    plsc.subcore_barrier()

    def idx_issue(blk, q):
        base = base0 + blk * K
        pltpu.async_copy(src_hbm.at[pl.ds(base, K)], src_i[q], isem[q])
        pltpu.async_copy(dst_hbm.at[pl.ds(base, K)], dst_i[q], isem[q])

    def idx_wait(q):
        pltpu.make_async_copy(src_hbm.at[pl.ds(0, K)], src_i[q], isem[q]).wait()
        pltpu.make_async_copy(dst_hbm.at[pl.ds(0, K)], dst_i[q], isem[q]).wait()

    def gather_issue(q, p):
        pltpu.async_copy(xl_hbm.at[src_i[q]], xl_rows[p], glsem[p])
        pltpu.async_copy(xr_hbm.at[dst_i[q]], xr_rows[p], grsem[p])

    def gather_wait(q, p):
        pltpu.make_async_copy(xl_hbm.at[src_i[q]], xl_rows[p], glsem[p]).wait()
        pltpu.make_async_copy(xr_hbm.at[dst_i[q]], xr_rows[p], grsem[p]).wait()

    def scatter_issue(q, sp):
        pltpu.async_copy(stage[sp], acc_sh.at[dst_i[q]], ssem[sp], add=True)

    def scatter_wait(q, sp):
        pltpu.make_async_copy(stage[sp], acc_sh.at[dst_i[q]], ssem[sp]).wait()

    def compute(p):
        @plsc.parallel_loop(0, K, unroll=4)
        def edge_body(j):
            acc = zero16
            xs = []
            for g2 in range(D // 32):
                a0 = xl_rows[p][j, pl.ds(32 * g2, 16)]
                a1 = xl_rows[p][j, pl.ds(32 * g2 + 16, 16)]
                xs.append(a0)
                xs.append(a1)
                u = xr_rows[p][j, pl.ds(16 * g2, 16)]
                b0 = lax.bitcast_convert_type(u << 16, jnp.float32)
                b1 = lax.bitcast_convert_type(u & jnp.int32(-65536), jnp.float32)
                m0 = a0 + b0
                m1 = a1 + b1
                lr0 = jnp.maximum(m0, 0.2 * m0)
                lr1 = jnp.maximum(m1, 0.2 * m1)
                acc = acc + attc[2 * g2] * lr0
                acc = acc + attc[2 * g2 + 1] * lr1
            for pp in range(4):
                acc = acc + acc.at[perm[pp]].get(mode="promise_in_bounds")
            exv = jnp.exp(acc)
            for g in range(D // 16):
                stage[p][j, pl.ds(16 * g, 16)] = xs[g] * exv
            stage[p][j, pl.ds(D, 16)] = jnp.where(lane0, exv, 0.0)

    idx_issue(0, 0)
    idx_issue(1, 1)
    idx_wait(0)
    gather_issue(0, 0)

    def group_body(t, _):
        for phi in range(4):
            blk = t * 4 + phi
            p = phi % 2
            q = phi % 4
            if phi < 2:
                @pl.when(t > 0)
                def _():
                    scatter_wait((phi - 2) % 4, p)
            else:
                scatter_wait(phi - 2, p)
            idx_issue(blk + 2, (phi + 2) % 4)
            gather_wait(q, p)
            compute(p)
            scatter_issue(q, p)
            idx_wait((phi + 1) % 4)
            gather_issue((phi + 1) % 4, (phi + 1) % 2)
        return 0
    lax.fori_loop(0, NBLK // 4, group_body, 0)

    scatter_wait((NBLK - 2) % 4, (NBLK - 2) % 2)
    scatter_wait((NBLK - 1) % 4, (NBLK - 1) % 2)
    gather_wait(NBLK % 4, NBLK % 2)
    idx_wait((NBLK + 1) % 4)

    plsc.subcore_barrier()
    for k in range(nfull):
        r = row0 + k * K
        pltpu.sync_copy(acc_sh.at[pl.ds(r, K)], out_hbm.at[c, pl.ds(r, K)])
    if rem:
        r = row0 + nfull * K
        pltpu.sync_copy(acc_sh.at[pl.ds(r, rem)], out_hbm.at[c, pl.ds(r, rem)])


_sc_conv = pl.kernel(
    _sc_conv_body,
    out_type=jax.ShapeDtypeStruct((NC, NP, AW), jnp.float32),
    mesh=plsc.VectorSubcoreMesh(core_axis_name="c", subcore_axis_name="s"),
    compiler_params=pltpu.CompilerParams(use_tc_tiling_on_sc=False),
    scratch_types=[
        [pltpu.VMEM((K,), jnp.int32)] * 4,
        [pltpu.VMEM((K,), jnp.int32)] * 4,
        [pltpu.VMEM((K, D), jnp.float32)] * 2,
        [pltpu.VMEM((K, D // 2), jnp.int32)] * 2,
        [pltpu.VMEM((K, AW), jnp.float32)] * 2,
        pltpu.VMEM((D,), jnp.float32),
        pltpu.VMEM_SHARED((NP, AW), jnp.float32),
        [pltpu.SemaphoreType.DMA] * 4,
        [pltpu.SemaphoreType.DMA] * 2,
        [pltpu.SemaphoreType.DMA] * 2,
        [pltpu.SemaphoreType.DMA] * 2,
    ],
)


_MM_ROWS = 1264


def _mm_block_body(x_ref, w_ref, *o_refs):
    r = jnp.dot(x_ref[...], w_ref[...], preferred_element_type=jnp.float32)
    for i, o in enumerate(o_refs):
        o[...] = r[:, i * D:(i + 1) * D].astype(o.dtype)


def _mm(xp, w, dtypes):
    nout = len(dtypes)
    grid = NP // _MM_ROWS
    return pl.pallas_call(
        _mm_block_body,
        grid=(grid,),
        in_specs=[
            pl.BlockSpec((_MM_ROWS, D), lambda i: (i, 0)),
            pl.BlockSpec((D, nout * D), lambda i: (0, 0)),
        ],
        out_specs=[pl.BlockSpec((_MM_ROWS, D), lambda i: (i, 0))] * nout,
        out_shape=[jax.ShapeDtypeStruct((NP, D), dt) for dt in dtypes],
    )(xp, w)


def _bn_from_acc(p, b, g, be, mask):
    m = p[0] + p[1]
    o = m[:, :D] / (m[:, D:D + 1] + 1e-16) + b
    o = jnp.where(mask, o, 0.0)
    mean = jnp.sum(o, axis=0, keepdims=True) * (1.0 / N)
    d = jnp.where(mask, o - mean, 0.0)
    var = jnp.sum(d * d, axis=0, keepdims=True) * (1.0 / N)
    return (o - mean) * lax.rsqrt(var + 1e-5) * g + be


def _epi1_body(p_ref, b_ref, g_ref, be_ref, o_ref):
    mask = lax.broadcasted_iota(jnp.int32, (NP, 1), 0) < N
    y = _bn_from_acc(p_ref[...], b_ref[...], g_ref[...], be_ref[...], mask)
    o_ref[...] = jnp.where(mask, jnp.maximum(y, 0.0), 0.0)


def _epif_body(p2_ref, b2_ref, g2_ref, be2_ref,
               p3_ref, b3_ref, g3_ref, be3_ref, o_ref):
    mask = lax.broadcasted_iota(jnp.int32, (NP, 1), 0) < N
    y2 = _bn_from_acc(p2_ref[...], b2_ref[...], g2_ref[...], be2_ref[...], mask)
    y3 = _bn_from_acc(p3_ref[...], b3_ref[...], g3_ref[...], be3_ref[...], mask)
    o_ref[...] = jnp.where(mask, jnp.maximum(y2 + y3, 0.0), 0.0)


def _epi1(acc, b, g, be):
    return pl.pallas_call(
        _epi1_body,
        out_shape=jax.ShapeDtypeStruct((NP, D), jnp.float32),
    )(acc, b.reshape(1, D), g.reshape(1, D), be.reshape(1, D))


def _epif(acc2, b2, g2, be2, acc3, b3, g3, be3):
    return pl.pallas_call(
        _epif_body,
        out_shape=jax.ShapeDtypeStruct((NP, D), jnp.float32),
    )(acc2, b2.reshape(1, D), g2.reshape(1, D), be2.reshape(1, D),
      acc3, b3.reshape(1, D), g3.reshape(1, D), be3.reshape(1, D))


def kernel(x, edge_index, Wl1, Wr1, att1, b1, g1, be1,
           Wl2, Wr2, att2, b2, g2, be2,
           Wl3, Wr3, att3, b3, g3, be3):
    f32 = jnp.float32
    xp = jnp.zeros((NP, D), f32).at[:N].set(x.astype(f32))

    src = edge_index[0].astype(jnp.int32)
    dst = edge_index[1].astype(jnp.int32)
    loop = jnp.arange(N, dtype=jnp.int32)
    padi = jnp.full((EPAD - EL,), N, jnp.int32)
    srcp = jnp.concatenate([src, loop, padi])
    dstp = jnp.concatenate([dst, loop, padi])

    sig = (jnp.arange(D // 32)[:, None, None] * 32
           + jnp.arange(16)[None, :, None]
           + jnp.arange(2)[None, None, :] * 16).reshape(-1)
    def _pack(xr_bf):
        return lax.bitcast_convert_type(
            xr_bf.reshape(NP, D // 2, 2), jnp.int32)

    f32b = (jnp.float32, jnp.bfloat16)
    w1 = jnp.concatenate([Wl1.T, Wr1[sig].T, Wl3.T, Wr3[sig].T], axis=1)
    xl1, xr1, xl3, xr3 = _mm(xp, w1, f32b + f32b)
    acc1 = _sc_conv(xl1, _pack(xr1), srcp, dstp, att1)
    h1 = _epi1(acc1, b1, g1, be1)

    w2 = jnp.concatenate([Wl2.T, Wr2[sig].T], axis=1)
    xl2, xr2 = _mm(h1, w2, f32b)
    acc2 = _sc_conv(xl2, _pack(xr2), srcp, dstp, att2)
    acc3 = _sc_conv(xl3, _pack(xr3), srcp, dstp, att3)

    y = _epif(acc2, b2, g2, be2, acc3, b3, g3, be3)
    return y[:N]


# speedup vs baseline: 1.2369x; 1.0804x over previous
import functools

import jax
import jax.numpy as jnp
from jax import lax
from jax.experimental import pallas as pl
from jax.experimental.pallas import tpu as pltpu, tpu_sc as plsc

N = 10000
D = 128
E = 320000
EL = E + N
NP = 10112
AW = 144
K = 64
NC = 2
NS = 16
NW = NC * NS
NBLK = 162
PER_TILE = NBLK * K
EP = PER_TILE * NW
EPAD = EP + 3 * K
ROWS_PER_TILE = NP // NS


def _sc_conv_body(xl_hbm, xr_hbm, src_hbm, dst_hbm, att_hbm, out_hbm,
                  src_i, dst_i, xl_rows, xr_rows, stage, att_v,
                  acc_sh, isem, glsem, grsem, ssem):
    c = lax.axis_index("c")
    s = lax.axis_index("s")
    wid = s * NC + c
    base0 = wid * PER_TILE

    pltpu.sync_copy(att_hbm, att_v)
    iota = lax.iota(jnp.int32, 16)
    lane0 = iota == 0
    perm = [jnp.bitwise_xor(iota, 1 << p) for p in range(4)]
    attc = [att_v[pl.ds(16 * g, 16)] for g in range(D // 16)]
    zero16 = jnp.zeros((16,), jnp.float32)

    @plsc.parallel_loop(0, K, unroll=4)
    def zrow(j):
        for g in range(AW // 16):
            stage[j, pl.ds(16 * g, 16)] = zero16
    row0 = s * ROWS_PER_TILE
    nfull = ROWS_PER_TILE // K
    for k in range(nfull):
        pltpu.sync_copy(stage, acc_sh.at[pl.ds(row0 + k * K, K)])
    rem = ROWS_PER_TILE - nfull * K
    if rem:
        pltpu.sync_copy(stage.at[pl.ds(0, rem)],
                        acc_sh.at[pl.ds(row0 + nfull * K, rem)])
    plsc.subcore_barrier()

    def idx_issue(blk, q):
        base = base0 + blk * K
        pltpu.async_copy(src_hbm.at[pl.ds(base, K)], src_i[q], isem[q])
        pltpu.async_copy(dst_hbm.at[pl.ds(base, K)], dst_i[q], isem[q])

    def idx_wait(q):
        pltpu.make_async_copy(src_hbm.at[pl.ds(0, K)], src_i[q], isem[q]).wait()
        pltpu.make_async_copy(dst_hbm.at[pl.ds(0, K)], dst_i[q], isem[q]).wait()

    def gather_issue(q, p):
        pltpu.async_copy(xl_hbm.at[src_i[q]], xl_rows[p], glsem[p])
        pltpu.async_copy(xr_hbm.at[dst_i[q]], xr_rows[p], grsem[p])

    def gather_wait(q, p):
        pltpu.make_async_copy(xl_hbm.at[src_i[q]], xl_rows[p], glsem[p]).wait()
        pltpu.make_async_copy(xr_hbm.at[dst_i[q]], xr_rows[p], grsem[p]).wait()

    def scatter_issue(q):
        pltpu.async_copy(stage, acc_sh.at[dst_i[q]], ssem, add=True)

    def scatter_wait(q):
        pltpu.make_async_copy(stage, acc_sh.at[dst_i[q]], ssem).wait()

    def compute(p):
        @plsc.parallel_loop(0, K, unroll=4)
        def edge_body(j):
            acc = zero16
            xs = []
            for g2 in range(D // 32):
                a0 = xl_rows[p][j, pl.ds(32 * g2, 16)]
                a1 = xl_rows[p][j, pl.ds(32 * g2 + 16, 16)]
                xs.append(a0)
                xs.append(a1)
                u = xr_rows[p][j, pl.ds(16 * g2, 16)]
                b0 = lax.bitcast_convert_type(u << 16, jnp.float32)
                b1 = lax.bitcast_convert_type(u & jnp.int32(-65536), jnp.float32)
                m0 = a0 + b0
                m1 = a1 + b1
                lr0 = jnp.maximum(m0, 0.2 * m0)
                lr1 = jnp.maximum(m1, 0.2 * m1)
                acc = acc + attc[2 * g2] * lr0
                acc = acc + attc[2 * g2 + 1] * lr1
            for pp in range(4):
                acc = acc + acc.at[perm[pp]].get(mode="promise_in_bounds")
            exv = jnp.exp(acc)
            for g in range(D // 16):
                stage[j, pl.ds(16 * g, 16)] = xs[g] * exv
            stage[j, pl.ds(D, 16)] = jnp.where(lane0, exv, 0.0)

    idx_issue(0, 0)
    idx_issue(1, 1)
    idx_wait(0)
    gather_issue(0, 0)

    def group_body(t, _):
        for phi in range(6):
            blk = t * 6 + phi
            p = phi % 2
            q = phi % 3
            if phi == 0:
                @pl.when(t > 0)
                def _():
                    scatter_wait(2)
            else:
                scatter_wait((phi - 1) % 3)
            idx_issue(blk + 2, (phi + 2) % 3)
            gather_wait(q, p)
            compute(p)
            scatter_issue(q)
            idx_wait((phi + 1) % 3)
            gather_issue((phi + 1) % 3, (phi + 1) % 2)
        return 0
    lax.fori_loop(0, NBLK // 6, group_body, 0)

    scatter_wait((NBLK - 1) % 3)
    gather_wait(NBLK % 3, NBLK % 2)
    idx_wait((NBLK + 1) % 3)

    plsc.subcore_barrier()
    for k in range(nfull):
        r = row0 + k * K
        pltpu.sync_copy(acc_sh.at[pl.ds(r, K)], out_hbm.at[c, pl.ds(r, K)])
    if rem:
        r = row0 + nfull * K
        pltpu.sync_copy(acc_sh.at[pl.ds(r, rem)], out_hbm.at[c, pl.ds(r, rem)])


_sc_conv = pl.kernel(
    _sc_conv_body,
    out_type=jax.ShapeDtypeStruct((NC, NP, AW), jnp.float32),
    mesh=plsc.VectorSubcoreMesh(core_axis_name="c", subcore_axis_name="s"),
    compiler_params=pltpu.CompilerParams(use_tc_tiling_on_sc=False),
    scratch_types=[
        [pltpu.VMEM((K,), jnp.int32)] * 3,
        [pltpu.VMEM((K,), jnp.int32)] * 3,
        [pltpu.VMEM((K, D), jnp.float32)] * 2,
        [pltpu.VMEM((K, D // 2), jnp.int32)] * 2,
        pltpu.VMEM((K, AW), jnp.float32),
        pltpu.VMEM((D,), jnp.float32),
        pltpu.VMEM_SHARED((NP, AW), jnp.float32),
        [pltpu.SemaphoreType.DMA] * 3,
        [pltpu.SemaphoreType.DMA] * 2,
        [pltpu.SemaphoreType.DMA] * 2,
        pltpu.SemaphoreType.DMA,
    ],
)


_MM_ROWS = 1264


def _mm_block_body(x_ref, w_ref, *o_refs):
    r = jnp.dot(x_ref[...], w_ref[...], preferred_element_type=jnp.float32)
    for i, o in enumerate(o_refs):
        o[...] = r[:, i * D:(i + 1) * D].astype(o.dtype)


def _mm(xp, w, dtypes):
    nout = len(dtypes)
    grid = NP // _MM_ROWS
    return pl.pallas_call(
        _mm_block_body,
        grid=(grid,),
        in_specs=[
            pl.BlockSpec((_MM_ROWS, D), lambda i: (i, 0)),
            pl.BlockSpec((D, nout * D), lambda i: (0, 0)),
        ],
        out_specs=[pl.BlockSpec((_MM_ROWS, D), lambda i: (i, 0))] * nout,
        out_shape=[jax.ShapeDtypeStruct((NP, D), dt) for dt in dtypes],
    )(xp, w)


def _bn_from_acc(p, b, g, be, mask):
    m = p[0] + p[1]
    o = m[:, :D] / (m[:, D:D + 1] + 1e-16) + b
    o = jnp.where(mask, o, 0.0)
    mean = jnp.sum(o, axis=0, keepdims=True) * (1.0 / N)
    d = jnp.where(mask, o - mean, 0.0)
    var = jnp.sum(d * d, axis=0, keepdims=True) * (1.0 / N)
    return (o - mean) * lax.rsqrt(var + 1e-5) * g + be


def _epi1_body(p_ref, b_ref, g_ref, be_ref, o_ref):
    mask = lax.broadcasted_iota(jnp.int32, (NP, 1), 0) < N
    y = _bn_from_acc(p_ref[...], b_ref[...], g_ref[...], be_ref[...], mask)
    o_ref[...] = jnp.where(mask, jnp.maximum(y, 0.0), 0.0)


def _epif_body(p2_ref, b2_ref, g2_ref, be2_ref,
               p3_ref, b3_ref, g3_ref, be3_ref, o_ref):
    mask = lax.broadcasted_iota(jnp.int32, (NP, 1), 0) < N
    y2 = _bn_from_acc(p2_ref[...], b2_ref[...], g2_ref[...], be2_ref[...], mask)
    y3 = _bn_from_acc(p3_ref[...], b3_ref[...], g3_ref[...], be3_ref[...], mask)
    o_ref[...] = jnp.where(mask, jnp.maximum(y2 + y3, 0.0), 0.0)


def _epi1(acc, b, g, be):
    return pl.pallas_call(
        _epi1_body,
        out_shape=jax.ShapeDtypeStruct((NP, D), jnp.float32),
    )(acc, b.reshape(1, D), g.reshape(1, D), be.reshape(1, D))


def _epif(acc2, b2, g2, be2, acc3, b3, g3, be3):
    return pl.pallas_call(
        _epif_body,
        out_shape=jax.ShapeDtypeStruct((NP, D), jnp.float32),
    )(acc2, b2.reshape(1, D), g2.reshape(1, D), be2.reshape(1, D),
      acc3, b3.reshape(1, D), g3.reshape(1, D), be3.reshape(1, D))


def kernel(x, edge_index, Wl1, Wr1, att1, b1, g1, be1,
           Wl2, Wr2, att2, b2, g2, be2,
           Wl3, Wr3, att3, b3, g3, be3):
    f32 = jnp.float32
    xp = jnp.zeros((NP, D), f32).at[:N].set(x.astype(f32))

    src = edge_index[0].astype(jnp.int32)
    dst = edge_index[1].astype(jnp.int32)
    loop = jnp.arange(N, dtype=jnp.int32)
    padi = jnp.full((EPAD - EL,), N, jnp.int32)
    srcp = jnp.concatenate([src, loop, padi])
    dstp = jnp.concatenate([dst, loop, padi])

    sig = (jnp.arange(D // 32)[:, None, None] * 32
           + jnp.arange(16)[None, :, None]
           + jnp.arange(2)[None, None, :] * 16).reshape(-1)
    def _pack(xr_bf):
        return lax.bitcast_convert_type(
            xr_bf.reshape(NP, D // 2, 2), jnp.int32)

    f32b = (jnp.float32, jnp.bfloat16)
    w1 = jnp.concatenate([Wl1.T, Wr1[sig].T, Wl3.T, Wr3[sig].T], axis=1)
    xl1, xr1, xl3, xr3 = _mm(xp, w1, f32b + f32b)
    acc1 = _sc_conv(xl1, _pack(xr1), srcp, dstp, att1)
    h1 = _epi1(acc1, b1, g1, be1)

    w2 = jnp.concatenate([Wl2.T, Wr2[sig].T], axis=1)
    xl2, xr2 = _mm(h1, w2, f32b)
    acc2 = _sc_conv(xl2, _pack(xr2), srcp, dstp, att2)
    acc3 = _sc_conv(xl3, _pack(xr3), srcp, dstp, att3)

    y = _epif(acc2, b2, g2, be2, acc3, b3, g3, be3)
    return y[:N]


# speedup vs baseline: 1.2372x; 1.0003x over previous
import jax
import jax.numpy as jnp
from jax import lax
from jax.experimental import pallas as pl
from jax.experimental.pallas import tpu as pltpu, tpu_sc as plsc

N = 10000
D = 128
E = 320000
EL = E + N
NP = 10112
AW = 144
K = 64
NC = 2
NS = 16
NW = NC * NS
NBLK = 162
PER_TILE = NBLK * K
EP = PER_TILE * NW
EPAD = EP + 3 * K
ROWS_PER_TILE = NP // NS


def _sc_conv_body(xl_hbm, xr_hbm, src_hbm, dst_hbm, att_hbm, out_hbm,
                  src_i, dst_i, xl_rows, xr_rows, stage, att_v,
                  acc_sh, isem, glsem, grsem, ssem):
    c = lax.axis_index("c")
    s = lax.axis_index("s")
    wid = s * NC + c
    base0 = wid * PER_TILE

    pltpu.sync_copy(att_hbm, att_v)
    iota = lax.iota(jnp.int32, 16)
    lane0 = iota == 0
    perm = [jnp.bitwise_xor(iota, 1 << p) for p in range(4)]
    attc = [att_v[pl.ds(16 * g, 16)] for g in range(D // 16)]
    zero16 = jnp.zeros((16,), jnp.float32)

    @plsc.parallel_loop(0, K, unroll=4)
    def zrow(j):
        for g in range(AW // 16):
            stage[j, pl.ds(16 * g, 16)] = zero16
    row0 = s * ROWS_PER_TILE
    nfull = ROWS_PER_TILE // K
    for k in range(nfull):
        pltpu.sync_copy(stage, acc_sh.at[pl.ds(row0 + k * K, K)])
    rem = ROWS_PER_TILE - nfull * K
    if rem:
        pltpu.sync_copy(stage.at[pl.ds(0, rem)],
                        acc_sh.at[pl.ds(row0 + nfull * K, rem)])
    plsc.subcore_barrier()

    def idx_issue(blk, q):
        base = base0 + blk * K
        pltpu.async_copy(src_hbm.at[pl.ds(base, K)], src_i[q], isem[q])
        pltpu.async_copy(dst_hbm.at[pl.ds(base, K)], dst_i[q], isem[q])

    def idx_wait(q):
        pltpu.make_async_copy(src_hbm.at[pl.ds(0, K)], src_i[q], isem[q]).wait()
        pltpu.make_async_copy(dst_hbm.at[pl.ds(0, K)], dst_i[q], isem[q]).wait()

    def gather_issue(q, p):
        pltpu.async_copy(xl_hbm.at[src_i[q]], xl_rows[p], glsem[p])
        pltpu.async_copy(xr_hbm.at[dst_i[q]], xr_rows[p], grsem[p])

    def gather_wait(q, p):
        pltpu.make_async_copy(xl_hbm.at[src_i[q]], xl_rows[p], glsem[p]).wait()
        pltpu.make_async_copy(xr_hbm.at[dst_i[q]], xr_rows[p], grsem[p]).wait()

    def scatter_issue(q):
        pltpu.async_copy(stage, acc_sh.at[dst_i[q]], ssem, add=True)

    def scatter_wait(q):
        pltpu.make_async_copy(stage, acc_sh.at[dst_i[q]], ssem).wait()

    def compute(p):
        @plsc.parallel_loop(0, K, unroll=4)
        def edge_body(j):
            acc = zero16
            xs = []
            for g2 in range(D // 32):
                a0 = xl_rows[p][j, pl.ds(32 * g2, 16)]
                a1 = xl_rows[p][j, pl.ds(32 * g2 + 16, 16)]
                xs.append(a0)
                xs.append(a1)
                u = xr_rows[p][j, pl.ds(16 * g2, 16)]
                b0 = lax.bitcast_convert_type(u << 16, jnp.float32)
                b1 = lax.bitcast_convert_type(u & jnp.int32(-65536), jnp.float32)
                m0 = a0 + b0
                m1 = a1 + b1
                lr0 = jnp.maximum(m0, 0.2 * m0)
                lr1 = jnp.maximum(m1, 0.2 * m1)
                acc = acc + attc[2 * g2] * lr0
                acc = acc + attc[2 * g2 + 1] * lr1
            for pp in range(4):
                acc = acc + acc.at[perm[pp]].get(mode="promise_in_bounds")
            exv = jnp.exp(acc)
            for g in range(D // 16):
                stage[j, pl.ds(16 * g, 16)] = xs[g] * exv
            stage[j, pl.ds(D, 16)] = jnp.where(lane0, exv, 0.0)

    idx_issue(0, 0)
    idx_issue(1, 1)
    idx_wait(0)
    gather_issue(0, 0)

    def group_body(t, _):
        for phi in range(6):
            blk = t * 6 + phi
            p = phi % 2
            q = phi % 3
            if phi == 0:
                @pl.when(t > 0)
                def _():
                    scatter_wait(2)
            else:
                scatter_wait((phi - 1) % 3)
            idx_issue(blk + 2, (phi + 2) % 3)
            gather_wait(q, p)
            compute(p)
            scatter_issue(q)
            idx_wait((phi + 1) % 3)
            gather_issue((phi + 1) % 3, (phi + 1) % 2)
        return 0
    lax.fori_loop(0, NBLK // 6, group_body, 0)

    scatter_wait((NBLK - 1) % 3)
    gather_wait(NBLK % 3, NBLK % 2)
    idx_wait((NBLK + 1) % 3)

    plsc.subcore_barrier()
    for k in range(nfull):
        r = row0 + k * K
        pltpu.sync_copy(acc_sh.at[pl.ds(r, K)], out_hbm.at[c, pl.ds(r, K)])
    if rem:
        r = row0 + nfull * K
        pltpu.sync_copy(acc_sh.at[pl.ds(r, rem)], out_hbm.at[c, pl.ds(r, rem)])


_sc_conv = pl.kernel(
    _sc_conv_body,
    out_type=jax.ShapeDtypeStruct((NC, NP, AW), jnp.float32),
    mesh=plsc.VectorSubcoreMesh(core_axis_name="c", subcore_axis_name="s"),
    compiler_params=pltpu.CompilerParams(use_tc_tiling_on_sc=False),
    scratch_types=[
        [pltpu.VMEM((K,), jnp.int32)] * 3,
        [pltpu.VMEM((K,), jnp.int32)] * 3,
        [pltpu.VMEM((K, D), jnp.float32)] * 2,
        [pltpu.VMEM((K, D // 2), jnp.int32)] * 2,
        pltpu.VMEM((K, AW), jnp.float32),
        pltpu.VMEM((D,), jnp.float32),
        pltpu.VMEM_SHARED((NP, AW), jnp.float32),
        [pltpu.SemaphoreType.DMA] * 3,
        [pltpu.SemaphoreType.DMA] * 2,
        [pltpu.SemaphoreType.DMA] * 2,
        pltpu.SemaphoreType.DMA,
    ],
)


_MM_ROWS = 1264


def _mm_block_body(x_ref, w_ref, *o_refs):
    r = jnp.dot(x_ref[...], w_ref[...], preferred_element_type=jnp.float32)
    for i, o in enumerate(o_refs):
        o[...] = r[:, i * D:(i + 1) * D].astype(o.dtype)


def _mm(xp, w, dtypes):
    nout = len(dtypes)
    grid = NP // _MM_ROWS
    return pl.pallas_call(
        _mm_block_body,
        grid=(grid,),
        in_specs=[
            pl.BlockSpec((_MM_ROWS, D), lambda i: (i, 0)),
            pl.BlockSpec((D, nout * D), lambda i: (0, 0)),
        ],
        out_specs=[pl.BlockSpec((_MM_ROWS, D), lambda i: (i, 0))] * nout,
        out_shape=[jax.ShapeDtypeStruct((NP, D), dt) for dt in dtypes],
    )(xp, w)


def _bn_from_acc(p, b, g, be, mask):
    m = p[0] + p[1]
    o = m[:, :D] / (m[:, D:D + 1] + 1e-16) + b
    o = jnp.where(mask, o, 0.0)
    mean = jnp.sum(o, axis=0, keepdims=True) * (1.0 / N)
    d = jnp.where(mask, o - mean, 0.0)
    var = jnp.sum(d * d, axis=0, keepdims=True) * (1.0 / N)
    return (o - mean) * lax.rsqrt(var + 1e-5) * g + be


def _epi1_body(p_ref, b_ref, g_ref, be_ref, o_ref):
    mask = lax.broadcasted_iota(jnp.int32, (NP, 1), 0) < N
    y = _bn_from_acc(p_ref[...], b_ref[...], g_ref[...], be_ref[...], mask)
    o_ref[...] = jnp.where(mask, jnp.maximum(y, 0.0), 0.0)


def _epif_body(p2_ref, b2_ref, g2_ref, be2_ref,
               p3_ref, b3_ref, g3_ref, be3_ref, o_ref):
    mask = lax.broadcasted_iota(jnp.int32, (NP, 1), 0) < N
    y2 = _bn_from_acc(p2_ref[...], b2_ref[...], g2_ref[...], be2_ref[...], mask)
    y3 = _bn_from_acc(p3_ref[...], b3_ref[...], g3_ref[...], be3_ref[...], mask)
    o_ref[...] = jnp.where(mask, jnp.maximum(y2 + y3, 0.0), 0.0)


def _epi1(acc, b, g, be):
    return pl.pallas_call(
        _epi1_body,
        out_shape=jax.ShapeDtypeStruct((NP, D), jnp.float32),
    )(acc, b.reshape(1, D), g.reshape(1, D), be.reshape(1, D))


def _epif(acc2, b2, g2, be2, acc3, b3, g3, be3):
    return pl.pallas_call(
        _epif_body,
        out_shape=jax.ShapeDtypeStruct((NP, D), jnp.float32),
    )(acc2, b2.reshape(1, D), g2.reshape(1, D), be2.reshape(1, D),
      acc3, b3.reshape(1, D), g3.reshape(1, D), be3.reshape(1, D))


def kernel(x, edge_index, Wl1, Wr1, att1, b1, g1, be1,
           Wl2, Wr2, att2, b2, g2, be2,
           Wl3, Wr3, att3, b3, g3, be3):
    f32 = jnp.float32
    xp = jnp.zeros((NP, D), f32).at[:N].set(x.astype(f32))

    src = edge_index[0].astype(jnp.int32)
    dst = edge_index[1].astype(jnp.int32)
    loop = jnp.arange(N, dtype=jnp.int32)
    padi = jnp.full((EPAD - EL,), N, jnp.int32)
    srcp = jnp.concatenate([src, loop, padi])
    dstp = jnp.concatenate([dst, loop, padi])

    sig = (jnp.arange(D // 32)[:, None, None] * 32
           + jnp.arange(16)[None, :, None]
           + jnp.arange(2)[None, None, :] * 16).reshape(-1)
    def _pack(xr_bf):
        return lax.bitcast_convert_type(
            xr_bf.reshape(NP, D // 2, 2), jnp.int32)

    f32b = (jnp.float32, jnp.bfloat16)
    w1 = jnp.concatenate([Wl1.T, Wr1[sig].T, Wl3.T, Wr3[sig].T], axis=1)
    xl1, xr1, xl3, xr3 = _mm(xp, w1, f32b + f32b)
    acc1 = _sc_conv(xl1, _pack(xr1), srcp, dstp, att1)
    h1 = _epi1(acc1, b1, g1, be1)

    w2 = jnp.concatenate([Wl2.T, Wr2[sig].T], axis=1)
    xl2, xr2 = _mm(h1, w2, f32b)
    acc2 = _sc_conv(xl2, _pack(xr2), srcp, dstp, att2)
    acc3 = _sc_conv(xl3, _pack(xr3), srcp, dstp, att3)

    y = _epif(acc2, b2, g2, be2, acc3, b3, g3, be3)
    return y[:N]
